# baseline (speedup 1.0000x reference)
import functools

import jax
import jax.numpy as jnp
from jax import lax
from jax.experimental import pallas as pl
from jax.experimental.pallas import tpu as pltpu
from jax.experimental.pallas import tpu_sc as plsc

B = 16384
D = 64
N_ITEMS = 1000000
L = 16
NC = 2
NS = 16
NW = NC * NS
CH = 512
NCH = 61
IPW = CH * NCH
TAIL0 = NW * IPW
TAILW = N_ITEMS - TAIL0
TAILB0 = N_ITEMS - 128
SLABW = CH + 11
HITCAP = 4096
WLCAP = 2048
NDUMP = NW
BPW = B // NW
PIECE = 4096


BK = 48
NBKT = 64
SENT = 0x7FFF << 14


def _bucket_lanes(ch, packed, chtmp, fill, bkt, iota, ones16):
    sch, spk = plsc.sort_key_val(ch, packed)
    chtmp[...] = sch
    prev = plsc.load_gather(chtmp, [jnp.maximum(iota - 1, 0)])
    boundary = (iota == 0) | (prev != sch)
    run_start = plsc.cummax(jnp.where(boundary, iota, 0))
    rank = iota - run_start
    sm = sch < (NBKT - 1)
    fillg = plsc.load_gather(fill, [sch])
    pos = sch * BK + jnp.minimum(fillg + rank, BK - 1)
    plsc.store_scatter(bkt, [pos], spk, mask=sm)
    plsc.addupdate_scatter(fill, [sch], ones16, mask=sm)


def _stage_body(u_hbm, i_hbm, tt_hbm, tail_hbm, gu_hbm, gi_hbm,
                slab0, slab1, pbuf, fill, bkt, chtmp, stage0, stage1,
                bu0, bu1, bi0, bi1, s0, s1, c0, c1):
    wid = lax.axis_index("s") * NC + lax.axis_index("c")
    dump = B + wid
    start = wid * IPW
    is_last = wid == NW - 1
    rw = jnp.where(is_last, IPW + TAILW, IPW)

    iota = lax.iota(jnp.int32, L)
    zero16 = jnp.zeros((L,), jnp.int32)
    ones16 = jnp.full((L,), 1, jnp.int32)

    slabs = (slab0, slab1)
    sems = (s0, s1)
    stages = (stage0, stage1)
    bus = (bu0, bu1)
    bis = (bi0, bi1)
    csems = (c0, c1)

    def issue(ch, par):
        off = pl.multiple_of(start + ch * CH, 128)
        pltpu.async_copy(
            tt_hbm.at[:, pl.ds(off, CH)], slabs[par].at[:, pl.ds(0, CH)],
            sems[par])

    def wait_slab(ch, par):
        off = pl.multiple_of(start + ch * CH, 128)
        pltpu.make_async_copy(
            tt_hbm.at[:, pl.ds(off, CH)], slabs[par].at[:, pl.ds(0, CH)],
            sems[par]).wait()

    issue(0, 0)
    issue(1, 1)

    for i in range(NBKT // L):
        fill[pl.ds(i * L, L)] = zero16
    sent_vec = jnp.full((L,), SENT, jnp.int32)
    for i in range(NBKT * BK // L):
        bkt[pl.ds(i * L, L)] = sent_vec

    def scan_piece(src_hbm, piece, flag):
        pltpu.sync_copy(src_hbm.at[pl.ds(piece * PIECE, PIECE)], pbuf)
        bbase = piece * PIECE

        def vec_step(vv, _):
            idx = pbuf[pl.ds(vv * L, L)]
            lo = idx - start
            m = (lo >= 0) & (lo < rw)
            chc = jnp.right_shift(jnp.clip(lo, 0, IPW + TAILW - 1), 9)
            ch = jnp.where(m, chc, NBKT - 1)
            bpos = bbase + vv * L + iota
            packed = bpos | (lo << 14) | (flag << 29)

            @pl.when(jnp.any(m))
            def _():
                _bucket_lanes(ch, packed, chtmp, fill, bkt, iota, ones16)
            return 0


        lax.fori_loop(0, PIECE // L, vec_step, 0)

    for piece in range(B // PIECE):
        scan_piece(u_hbm, piece, 0)
    for piece in range(B // PIECE):
        scan_piece(i_hbm, piece, 1)

    def process(bucket, window_lo, window_w, slab_rel, slab_w, par):
        slab = slabs[par]
        stage = stages[par]
        bu_v = bus[par]
        bi_v = bis[par]
        for g in range(BK // L):
            e = bkt[pl.ds(bucket * BK + g * L, L)]
            lo = jnp.right_shift(e, 14) & 0x7FFF
            bpos = e & 0x3FFF
            flag = jnp.right_shift(e, 29) & 1
            m = (lo >= window_lo) & (lo < window_lo + window_w)
            loc = jnp.clip(lo - slab_rel, 0, slab_w - 1)
            row = g * L + iota
            bu_v[pl.ds(g * L, L)] = jnp.where(m & (flag == 0), bpos, dump)
            bi_v[pl.ds(g * L, L)] = jnp.where(m & (flag == 1), bpos, dump)

            @pl.when(jnp.any(m))
            def _():
                def dim_step(c4, _):
                    for cc in range(4):
                        cvec = c4 * 4 + cc + zero16
                        vc = plsc.load_gather(slab, [cvec, loc])
                        plsc.store_scatter(stage, [row, cvec], vc)
                    return 0

                lax.fori_loop(0, D // 4, dim_step, 0)

    def fire(par):
        pltpu.async_copy(stages[par].at[:, pl.ds(0, 2 * D)],
                         gu_hbm.at[bus[par]], csems[par])
        pltpu.async_copy(stages[par].at[:, pl.ds(0, 2 * D)],
                         gi_hbm.at[bis[par]], csems[par])

    def drain(par):
        pltpu.make_async_copy(stages[par].at[:, pl.ds(0, 2 * D)],
                              gu_hbm.at[bus[par]], csems[par]).wait()
        pltpu.make_async_copy(stages[par].at[:, pl.ds(0, 2 * D)],
                              gi_hbm.at[bis[par]], csems[par]).wait()

    def do_chunk(ch, par, head):
        wait_slab(ch, par)
        if not head:
            drain(par)
        rel = ch * CH
        process(ch, rel, CH, rel, CH, par)
        fire(par)

    do_chunk(0, 0, True)
    issue(2, 0)
    do_chunk(1, 1, True)
    issue(3, 1)

    def chunk_iter(it, _):
        for par in range(2):
            ch = it * 2 + par
            do_chunk(ch, par, False)
            issue(ch + 2, par)
        return 0

    lax.fori_loop(1, (NCH - 3) // 2, chunk_iter, 0)
    do_chunk(NCH - 3, 0, False)
    issue(NCH - 1, 0)
    do_chunk(NCH - 2, 1, False)
    do_chunk(NCH - 1, 0, False)

    drain(1)
    off1 = pl.multiple_of(TAIL0, 128)
    pltpu.sync_copy(tt_hbm.at[:, pl.ds(off1, CH)],
                    slabs[0].at[:, pl.ds(0, CH)])
    drain(0)
    process(NCH, NCH * CH, CH, NCH * CH, CH, 0)
    fire(0)
    pltpu.sync_copy(tail_hbm, slabs[1].at[:, pl.ds(0, 128)])
    tb0 = TAILB0 - (NW - 1) * IPW
    drain(0)
    process(NCH + 1, tb0 + 64, 64, tb0, 128, 1)
    fire(1)
    drain(1)


TR = 17


HB = BPW // 2


def _combine_body(gu_hbm, gi_hbm, w_hbm, b_hbm, out_hbm,
                  gu_v, gi_v, w_v, b_v, tr, out_v):
    wid = lax.axis_index("s") * NC + lax.axis_index("c")
    base = wid * BPW
    pltpu.sync_copy(w_hbm.at[0], w_v)
    pltpu.sync_copy(b_hbm, b_v)

    w_chunks = [w_v[pl.ds(k * L, L)] for k in range(D // L)]
    b_vec = b_v[...]
    lane = lax.iota(jnp.int32, L)

    def half(h, _):
        hbase = base + h * HB
        pltpu.sync_copy(gu_hbm.at[pl.ds(hbase, HB)], gu_v)
        pltpu.sync_copy(gi_hbm.at[pl.ds(hbase, HB)], gi_v)

        def group(g, _):
            for j in range(L):
                e = g * L + j
                p = jnp.zeros((L,), jnp.float32)
                for k in range(D // L):
                    pu = gu_v[e, pl.ds(k * L, L)]
                    pi = gi_v[e, pl.ds(k * L, L)]
                    p = p + (pu * pi) * w_chunks[k]
                plsc.store_scatter(tr, [lane * TR + j], p)
            acc = b_vec
            for k in range(L):
                acc = acc + tr[pl.ds(k * TR, L)]
            out_v[pl.ds(g * L, L)] = 1.0 / (1.0 + jnp.exp(-acc))
            return 0

        lax.fori_loop(0, HB // L, group, 0)
        pltpu.sync_copy(out_v, out_hbm.at[pl.ds(hbase, HB)])
        return 0

    lax.fori_loop(0, 2, half, 0)


@jax.jit
def _gmf(u_input, i_input, table_t, tail_t, W, b16):
    mesh = plsc.VectorSubcoreMesh(core_axis_name="c", subcore_axis_name="s")
    stage_fn = functools.partial(
        pl.kernel,
        mesh=mesh,
        compiler_params=pltpu.CompilerParams(needs_layout_passes=False),
        out_type=(jax.ShapeDtypeStruct((B + NW, 2 * D), jnp.float32),
                  jax.ShapeDtypeStruct((B + NW, 2 * D), jnp.float32)),
        scratch_types=[
            pltpu.VMEM((D, SLABW), jnp.float32),
            pltpu.VMEM((D, SLABW), jnp.float32),
            pltpu.VMEM((PIECE,), jnp.int32),
            pltpu.VMEM((NBKT,), jnp.int32),
            pltpu.VMEM((NBKT * BK,), jnp.int32),
            pltpu.VMEM((L,), jnp.int32),
            pltpu.VMEM((BK, 2 * D + 1), jnp.float32),
            pltpu.VMEM((BK, 2 * D + 1), jnp.float32),
            pltpu.VMEM((BK,), jnp.int32),
            pltpu.VMEM((BK,), jnp.int32),
            pltpu.VMEM((BK,), jnp.int32),
            pltpu.VMEM((BK,), jnp.int32),
            pltpu.SemaphoreType.DMA,
            pltpu.SemaphoreType.DMA,
            pltpu.SemaphoreType.DMA,
            pltpu.SemaphoreType.DMA,
        ],
    )(_stage_body)
    gu, gi = stage_fn(u_input, i_input, table_t, tail_t)

    combine_fn = functools.partial(
        pl.kernel,
        mesh=mesh,
        compiler_params=pltpu.CompilerParams(needs_layout_passes=False),
        out_type=jax.ShapeDtypeStruct((B,), jnp.float32),
        scratch_types=[
            pltpu.VMEM((BPW // 2, 2 * D), jnp.float32),
            pltpu.VMEM((BPW // 2, 2 * D), jnp.float32),
            pltpu.VMEM((D,), jnp.float32),
            pltpu.VMEM((L,), jnp.float32),
            pltpu.VMEM((L * TR,), jnp.float32),
            pltpu.VMEM((BPW // 2,), jnp.float32),
        ],
    )(_combine_body)
    return combine_fn(gu, gi, W, b16)


def kernel(u_input, i_input, item_table, W, b):
    u32 = u_input.astype(jnp.int32)
    i32 = i_input.astype(jnp.int32)
    b16 = jnp.broadcast_to(b.astype(jnp.float32), (L,))
    tail_t = item_table.T[:, TAILB0:]
    return _gmf(u32, i32, item_table.T, tail_t, W, b16)


# speedup vs baseline: 1.0594x; 1.0594x over previous
import functools

import jax
import jax.numpy as jnp
from jax import lax
from jax.experimental import pallas as pl
from jax.experimental.pallas import tpu as pltpu
from jax.experimental.pallas import tpu_sc as plsc

B = 16384
D = 64
N_ITEMS = 1000000
L = 16
NC = 2
NS = 16
NW = NC * NS
CH = 512
NCH = 61
IPW = CH * NCH
TAIL0 = NW * IPW
TAILW = N_ITEMS - TAIL0
TAILB0 = N_ITEMS - 128
SLABW = CH + 11
NDUMP = NW
BPW = B // NW
PIECE = 4096


BK = 48
NBKT = 64
SENT = 0x7FFF << 14


def _bucket_lanes(ch, packed, chtmp, fill, bkt, iota, ones16):
    sch, spk = plsc.sort_key_val(ch, packed)
    chtmp[...] = sch
    prev = plsc.load_gather(chtmp, [jnp.maximum(iota - 1, 0)])
    boundary = (iota == 0) | (prev != sch)
    run_start = plsc.cummax(jnp.where(boundary, iota, 0))
    rank = iota - run_start
    sm = sch < (NBKT - 1)
    fillg = plsc.load_gather(fill, [sch])
    pos = sch * BK + jnp.minimum(fillg + rank, BK - 1)
    plsc.store_scatter(bkt, [pos], spk, mask=sm)
    plsc.addupdate_scatter(fill, [sch], ones16, mask=sm)


def _stage_body(u_hbm, i_hbm, tt_hbm, tail_hbm, gu_hbm, gi_hbm,
                slab0, slab1, pbuf, fill, bkt, chtmp, stage0, stage1,
                bu0, bu1, bi0, bi1, s0, s1, c0, c1):
    wid = lax.axis_index("s") * NC + lax.axis_index("c")
    dump = B + wid
    start = wid * IPW
    is_last = wid == NW - 1
    rw = jnp.where(is_last, IPW + TAILW, IPW)

    iota = lax.iota(jnp.int32, L)
    zero16 = jnp.zeros((L,), jnp.int32)
    ones16 = jnp.full((L,), 1, jnp.int32)

    slabs = (slab0, slab1)
    sems = (s0, s1)
    stages = (stage0, stage1)
    bus = (bu0, bu1)
    bis = (bi0, bi1)
    csems = (c0, c1)

    def issue(ch, par):
        off = pl.multiple_of(start + ch * CH, 128)
        pltpu.async_copy(
            tt_hbm.at[:, pl.ds(off, CH)], slabs[par].at[:, pl.ds(0, CH)],
            sems[par])

    def wait_slab(ch, par):
        off = pl.multiple_of(start + ch * CH, 128)
        pltpu.make_async_copy(
            tt_hbm.at[:, pl.ds(off, CH)], slabs[par].at[:, pl.ds(0, CH)],
            sems[par]).wait()

    issue(0, 0)
    issue(1, 1)

    for i in range(NBKT // L):
        fill[pl.ds(i * L, L)] = zero16
    sent_vec = jnp.full((L,), SENT, jnp.int32)
    for i in range(NBKT * BK // L):
        bkt[pl.ds(i * L, L)] = sent_vec

    def scan_piece(src_hbm, piece, flag):
        pltpu.sync_copy(src_hbm.at[pl.ds(piece * PIECE, PIECE)], pbuf)
        bbase = piece * PIECE

        def vec_step(vv, _):
            idx = pbuf[pl.ds(vv * L, L)]
            lo = idx - start
            m = (lo >= 0) & (lo < rw)
            chc = jnp.right_shift(jnp.clip(lo, 0, IPW + TAILW - 1), 9)
            ch = jnp.where(m, chc, NBKT - 1)
            bpos = bbase + vv * L + iota
            packed = bpos | (lo << 14) | (flag << 29)

            _bucket_lanes(ch, packed, chtmp, fill, bkt, iota, ones16)
            return 0


        lax.fori_loop(0, PIECE // L, vec_step, 0)

    for piece in range(B // PIECE):
        scan_piece(u_hbm, piece, 0)
    for piece in range(B // PIECE):
        scan_piece(i_hbm, piece, 1)

    def process(bucket, window_lo, window_w, slab_rel, slab_w, par):
        slab = slabs[par]
        stage = stages[par]
        bu_v = bus[par]
        bi_v = bis[par]
        for g in range(BK // L):
            e = bkt[pl.ds(bucket * BK + g * L, L)]
            lo = jnp.right_shift(e, 14) & 0x7FFF
            bpos = e & 0x3FFF
            flag = jnp.right_shift(e, 29) & 1
            m = (lo >= window_lo) & (lo < window_lo + window_w)
            loc = jnp.clip(lo - slab_rel, 0, slab_w - 1)
            row = g * L + iota
            bu_v[pl.ds(g * L, L)] = jnp.where(m & (flag == 0), bpos, dump)
            bi_v[pl.ds(g * L, L)] = jnp.where(m & (flag == 1), bpos, dump)

            @pl.when(jnp.any(m))
            def _():
                def dim_step(c4, _):
                    for cc in range(4):
                        cvec = c4 * 4 + cc + zero16
                        vc = plsc.load_gather(slab, [cvec, loc])
                        plsc.store_scatter(stage, [row, cvec], vc)
                    return 0

                lax.fori_loop(0, D // 4, dim_step, 0)

    def fire(par):
        pltpu.async_copy(stages[par].at[:, pl.ds(0, 2 * D)],
                         gu_hbm.at[bus[par]], csems[par])
        pltpu.async_copy(stages[par].at[:, pl.ds(0, 2 * D)],
                         gi_hbm.at[bis[par]], csems[par])

    def drain(par):
        pltpu.make_async_copy(stages[par].at[:, pl.ds(0, 2 * D)],
                              gu_hbm.at[bus[par]], csems[par]).wait()
        pltpu.make_async_copy(stages[par].at[:, pl.ds(0, 2 * D)],
                              gi_hbm.at[bis[par]], csems[par]).wait()

    def do_chunk(ch, par, head):
        wait_slab(ch, par)
        if not head:
            drain(par)
        rel = ch * CH
        process(ch, rel, CH, rel, CH, par)
        fire(par)

    do_chunk(0, 0, True)
    issue(2, 0)
    do_chunk(1, 1, True)
    issue(3, 1)

    def chunk_iter(it, _):
        for par in range(2):
            ch = it * 2 + par
            do_chunk(ch, par, False)
            issue(ch + 2, par)
        return 0

    lax.fori_loop(1, (NCH - 3) // 2, chunk_iter, 0)
    do_chunk(NCH - 3, 0, False)
    issue(NCH - 1, 0)
    do_chunk(NCH - 2, 1, False)
    do_chunk(NCH - 1, 0, False)

    drain(1)
    off1 = pl.multiple_of(TAIL0, 128)
    pltpu.sync_copy(tt_hbm.at[:, pl.ds(off1, CH)],
                    slabs[0].at[:, pl.ds(0, CH)])
    drain(0)
    process(NCH, NCH * CH, CH, NCH * CH, CH, 0)
    fire(0)
    pltpu.sync_copy(tail_hbm, slabs[1].at[:, pl.ds(0, 128)])
    tb0 = TAILB0 - (NW - 1) * IPW
    drain(0)
    process(NCH + 1, tb0 + 64, 64, tb0, 128, 1)
    fire(1)
    drain(1)


TR = 17


HB = BPW // 2


def _combine_body(gu_hbm, gi_hbm, w_hbm, b_hbm, out_hbm,
                  gu_v, gi_v, w_v, b_v, tr, out_v):
    wid = lax.axis_index("s") * NC + lax.axis_index("c")
    base = wid * BPW
    pltpu.sync_copy(w_hbm.at[0], w_v)
    pltpu.sync_copy(b_hbm, b_v)

    w_chunks = [w_v[pl.ds(k * L, L)] for k in range(D // L)]
    b_vec = b_v[...]
    lane = lax.iota(jnp.int32, L)

    def half(h, _):
        hbase = base + h * HB
        pltpu.sync_copy(gu_hbm.at[pl.ds(hbase, HB)], gu_v)
        pltpu.sync_copy(gi_hbm.at[pl.ds(hbase, HB)], gi_v)

        def group(g, _):
            for j in range(L):
                e = g * L + j
                p = jnp.zeros((L,), jnp.float32)
                for k in range(D // L):
                    pu = gu_v[e, pl.ds(k * L, L)]
                    pi = gi_v[e, pl.ds(k * L, L)]
                    p = p + (pu * pi) * w_chunks[k]
                plsc.store_scatter(tr, [lane * TR + j], p)
            acc = b_vec
            for k in range(L):
                acc = acc + tr[pl.ds(k * TR, L)]
            out_v[pl.ds(g * L, L)] = 1.0 / (1.0 + jnp.exp(-acc))
            return 0

        lax.fori_loop(0, HB // L, group, 0)
        pltpu.sync_copy(out_v, out_hbm.at[pl.ds(hbase, HB)])
        return 0

    lax.fori_loop(0, 2, half, 0)


@jax.jit
def _gmf(u_input, i_input, table_t, tail_t, W, b16):
    mesh = plsc.VectorSubcoreMesh(core_axis_name="c", subcore_axis_name="s")
    stage_fn = functools.partial(
        pl.kernel,
        mesh=mesh,
        compiler_params=pltpu.CompilerParams(needs_layout_passes=False),
        out_type=(jax.ShapeDtypeStruct((B + NW, 2 * D), jnp.float32),
                  jax.ShapeDtypeStruct((B + NW, 2 * D), jnp.float32)),
        scratch_types=[
            pltpu.VMEM((D, SLABW), jnp.float32),
            pltpu.VMEM((D, SLABW), jnp.float32),
            pltpu.VMEM((PIECE,), jnp.int32),
            pltpu.VMEM((NBKT,), jnp.int32),
            pltpu.VMEM((NBKT * BK,), jnp.int32),
            pltpu.VMEM((L,), jnp.int32),
            pltpu.VMEM((BK, 2 * D + 1), jnp.float32),
            pltpu.VMEM((BK, 2 * D + 1), jnp.float32),
            pltpu.VMEM((BK,), jnp.int32),
            pltpu.VMEM((BK,), jnp.int32),
            pltpu.VMEM((BK,), jnp.int32),
            pltpu.VMEM((BK,), jnp.int32),
            pltpu.SemaphoreType.DMA,
            pltpu.SemaphoreType.DMA,
            pltpu.SemaphoreType.DMA,
            pltpu.SemaphoreType.DMA,
        ],
    )(_stage_body)
    gu, gi = stage_fn(u_input, i_input, table_t, tail_t)

    combine_fn = functools.partial(
        pl.kernel,
        mesh=mesh,
        compiler_params=pltpu.CompilerParams(needs_layout_passes=False),
        out_type=jax.ShapeDtypeStruct((B,), jnp.float32),
        scratch_types=[
            pltpu.VMEM((BPW // 2, 2 * D), jnp.float32),
            pltpu.VMEM((BPW // 2, 2 * D), jnp.float32),
            pltpu.VMEM((D,), jnp.float32),
            pltpu.VMEM((L,), jnp.float32),
            pltpu.VMEM((L * TR,), jnp.float32),
            pltpu.VMEM((BPW // 2,), jnp.float32),
        ],
    )(_combine_body)
    return combine_fn(gu, gi, W, b16)


def kernel(u_input, i_input, item_table, W, b):
    u32 = u_input.astype(jnp.int32)
    i32 = i_input.astype(jnp.int32)
    b16 = jnp.broadcast_to(b.astype(jnp.float32), (L,))
    tail_t = item_table.T[:, TAILB0:]
    return _gmf(u32, i32, item_table.T, tail_t, W, b16)
